# Initial kernel scaffold; baseline (speedup 1.0000x reference)
import jax
import jax.numpy as jnp
from jax.experimental import pallas as pl


def kernel(points, reg_ranges, gray_ranges, strides, gt_bboxes, gt_labels):
    raise NotImplementedError("write your pallas kernel here")


# speedup vs baseline: 11.4360x; 11.4360x over previous
import jax
import jax.numpy as jnp
from jax.experimental import pallas as pl
from jax.experimental.pallas import tpu as pltpu

NUM_CLASSES = 80
PB = 1984


def _assign_kernel(pts_ref, rr_ref, gr_ref, st_ref, gtb_ref, lr_ref, lc_ref,
                   cls_ref, reg_ref):
    pts = pts_ref[...]
    px = pts[:, 0:1]
    py = pts[:, 1:2]
    gb = gtb_ref[0]
    gx = gb[0:1, :]
    gy = gb[1:2, :]
    gw = gb[2:3, :]
    gh = gb[3:4, :]

    d1 = px - gx
    d2 = py - gy
    d3 = (gx + gw - 1.0) - px
    d4 = (gy + gh - 1.0) - py
    hit = (jnp.minimum(jnp.minimum(d1, d2), jnp.minimum(d3, d4)) >= 0.0)
    hf = hit.astype(jnp.float32)
    f1 = d1 * hf
    f2 = d2 * hf
    f3 = d3 * hf
    f4 = d4 * hf
    lr_min = jnp.minimum(f1, f3)
    lr_max = jnp.maximum(f1, f3)
    tb_min = jnp.minimum(f2, f4)
    tb_max = jnp.maximum(f2, f4)
    scores = (jnp.maximum(lr_min, 0.0) / jnp.maximum(lr_max, 0.01)) * \
             (jnp.maximum(tb_min, 0.0) / jnp.maximum(tb_max, 0.01))
    scores = jnp.sqrt(scores)

    cx = gx + gw * 0.5
    cy = gy + gh * 0.5
    s2 = st_ref[...] * 0.5
    inside_core = (px >= cx - s2) & (px <= cx + s2) & \
                  (py >= cy - s2) & (py <= cy + s2) & hit
    scores = jnp.where(inside_core, 1.0, scores)

    measure = jnp.maximum(gw, gh)
    rr = rr_ref[...]
    gr = gr_ref[...]
    rlo = rr[:, 0:1]
    rhi = rr[:, 1:2]
    glo = gr[:, 0:1]
    ghi = gr[:, 1:2]
    green = (rlo <= measure) & (measure <= rhi) & hit
    gray = (((glo <= measure) & (measure < rlo)) |
            ((rhi < measure) & (measure <= ghi))) & hit
    scores = scores * green.astype(jnp.float32)

    G = scores.shape[1]
    max_scores = jnp.max(scores, axis=1, keepdims=True)
    gidx = jax.lax.broadcasted_iota(jnp.int32, scores.shape, 1)
    midx = jnp.min(jnp.where(scores == max_scores, gidx, G),
                   axis=1, keepdims=True)
    sel = gidx == midx

    labels_row = lr_ref[0]
    matched = jnp.sum(jnp.where(sel, labels_row, 0),
                      axis=1, keepdims=True)
    pos = max_scores > 0.0
    val = jnp.where(pos, max_scores, 0.0)

    ciota = jax.lax.broadcasted_iota(jnp.int32, (pts.shape[0], NUM_CLASSES), 1)
    cls = jnp.where(ciota == matched, val, 0.0)

    labels_col = lc_ref[0]
    giota = jax.lax.broadcasted_iota(jnp.int32, (G, NUM_CLASSES), 1)
    onehot = (giota == labels_col).astype(jnp.float32)
    gray_c = jnp.dot(gray.astype(jnp.float32), onehot,
                     preferred_element_type=jnp.float32) > 0.0
    cls = jnp.where(gray_c & (cls == 0.0), -1.0, cls)
    cls_ref[0] = cls

    posf = pos.astype(jnp.float32)
    r1 = jnp.sum(jnp.where(sel, d1, 0.0), axis=1, keepdims=True)
    r2 = jnp.sum(jnp.where(sel, d2, 0.0), axis=1, keepdims=True)
    r3 = jnp.sum(jnp.where(sel, d3, 0.0), axis=1, keepdims=True)
    r4 = jnp.sum(jnp.where(sel, d4, 0.0), axis=1, keepdims=True)
    reg_ref[0] = jnp.concatenate([r1, r2, r3, r4], axis=1) * posf


def kernel(points, reg_ranges, gray_ranges, strides, gt_bboxes, gt_labels):
    P = points.shape[0]
    B, G, _ = gt_bboxes.shape
    assert P % PB == 0, P
    npb = P // PB

    strides2 = strides.reshape(P, 1)
    gtb_t = jnp.transpose(gt_bboxes, (0, 2, 1))
    labels_r = gt_labels.reshape(B, 1, G)
    labels_c = gt_labels.reshape(B, G, 1)

    cls, reg = pl.pallas_call(
        _assign_kernel,
        grid=(B, npb),
        in_specs=[
            pl.BlockSpec((PB, 2), lambda b, i: (i, 0)),
            pl.BlockSpec((PB, 2), lambda b, i: (i, 0)),
            pl.BlockSpec((PB, 2), lambda b, i: (i, 0)),
            pl.BlockSpec((PB, 1), lambda b, i: (i, 0)),
            pl.BlockSpec((1, 4, G), lambda b, i: (b, 0, 0)),
            pl.BlockSpec((1, 1, G), lambda b, i: (b, 0, 0)),
            pl.BlockSpec((1, G, 1), lambda b, i: (b, 0, 0)),
        ],
        out_specs=[
            pl.BlockSpec((1, PB, NUM_CLASSES), lambda b, i: (b, i, 0)),
            pl.BlockSpec((1, PB, 4), lambda b, i: (b, i, 0)),
        ],
        out_shape=[
            jax.ShapeDtypeStruct((B, P, NUM_CLASSES), jnp.float32),
            jax.ShapeDtypeStruct((B, P, 4), jnp.float32),
        ],
        compiler_params=pltpu.CompilerParams(
            dimension_semantics=("parallel", "arbitrary"),
        ),
    )(points, reg_ranges, gray_ranges, strides2, gtb_t, labels_r, labels_c)
    return cls, reg


# speedup vs baseline: 16.9589x; 1.4829x over previous
import jax
import jax.numpy as jnp
from jax.experimental import pallas as pl
from jax.experimental.pallas import tpu as pltpu

NUM_CLASSES = 80
PB = 1984
LG = 128


def _assign_kernel(pts_ref, rr_ref, gr_ref, st_ref, gt_ref, gm_ref, oh_ref,
                   cls_ref, reg_ref):
    pts = pts_ref[...]
    px = pts[:, 0:1]
    py = pts[:, 1:2]
    g = gt_ref[0]
    gx = g[0:1, :]
    gy = g[1:2, :]
    gxe = g[2:3, :]
    gye = g[3:4, :]
    cx = g[4:5, :]
    cy = g[5:6, :]
    meas = g[6:7, :]

    d1 = px - gx
    d2 = py - gy
    d3 = gxe - px
    d4 = gye - py
    hit = jnp.minimum(jnp.minimum(d1, d2), jnp.minimum(d3, d4)) >= 0.0
    hf = hit.astype(jnp.float32)
    f1 = d1 * hf
    f2 = d2 * hf
    f3 = d3 * hf
    f4 = d4 * hf
    q = (jnp.maximum(jnp.minimum(f1, f3), 0.0) /
         jnp.maximum(jnp.maximum(f1, f3), 0.01)) * \
        (jnp.maximum(jnp.minimum(f2, f4), 0.0) /
         jnp.maximum(jnp.maximum(f2, f4), 0.01))

    s2 = st_ref[...] * 0.5
    inside_core = (px >= cx - s2) & (px <= cx + s2) & \
                  (py >= cy - s2) & (py <= cy + s2) & hit
    q = jnp.where(inside_core, 1.0, q)

    rr = rr_ref[...]
    gr = gr_ref[...]
    rlo = rr[:, 0:1]
    rhi = rr[:, 1:2]
    glo = gr[:, 0:1]
    ghi = gr[:, 1:2]
    green = (rlo <= meas) & (meas <= rhi) & hit
    gray = (((glo <= meas) & (meas < rlo)) |
            ((rhi < meas) & (meas <= ghi))) & hit
    q = q * green.astype(jnp.float32)

    PBn = q.shape[0]
    G = LG // 2
    qA = q[:, 0:G]
    qB = q[:, G:LG]
    maxA = jnp.max(qA, axis=1, keepdims=True)
    maxB = jnp.max(qB, axis=1, keepdims=True)
    maxfull = jnp.concatenate(
        [jnp.broadcast_to(maxA, (PBn, G)), jnp.broadcast_to(maxB, (PBn, G))],
        axis=1)
    gidx = jnp.bitwise_and(
        jax.lax.broadcasted_iota(jnp.int32, (PBn, LG), 1), G - 1)
    cand = jnp.where(q == maxfull, gidx, G)
    midxA = jnp.min(cand[:, 0:G], axis=1, keepdims=True)
    midxB = jnp.min(cand[:, G:LG], axis=1, keepdims=True)
    midxfull = jnp.concatenate(
        [jnp.broadcast_to(midxA, (PBn, G)), jnp.broadcast_to(midxB, (PBn, G))],
        axis=1)
    sel = (gidx == midxfull).astype(jnp.float32)

    t = jnp.dot(sel, gm_ref[0], preferred_element_type=jnp.float32)
    gcv = jnp.dot(gray.astype(jnp.float32), oh_ref[0],
                  preferred_element_type=jnp.float32)

    ciota = jax.lax.broadcasted_iota(jnp.int32, (PBn, NUM_CLASSES), 1)

    posA = maxA > 0.0
    valA = jnp.where(posA, jnp.sqrt(maxA), 0.0)
    matchedA = t[:, 4:5].astype(jnp.int32)
    clsA = jnp.where(ciota == matchedA, valA, 0.0)
    grayA = gcv[:, 0:NUM_CLASSES] > 0.0
    clsA = jnp.where(grayA & (clsA == 0.0), -1.0, clsA)
    cls_ref[0] = clsA
    regA = jnp.concatenate(
        [px - t[:, 0:1], py - t[:, 1:2], t[:, 2:3] - px, t[:, 3:4] - py],
        axis=1) * posA.astype(jnp.float32)
    reg_ref[0] = regA

    posB = maxB > 0.0
    valB = jnp.where(posB, jnp.sqrt(maxB), 0.0)
    matchedB = t[:, 12:13].astype(jnp.int32)
    clsB = jnp.where(ciota == matchedB, valB, 0.0)
    grayB = gcv[:, 128:128 + NUM_CLASSES] > 0.0
    clsB = jnp.where(grayB & (clsB == 0.0), -1.0, clsB)
    cls_ref[1] = clsB
    regB = jnp.concatenate(
        [px - t[:, 8:9], py - t[:, 9:10], t[:, 10:11] - px, t[:, 11:12] - py],
        axis=1) * posB.astype(jnp.float32)
    reg_ref[1] = regB


def kernel(points, reg_ranges, gray_ranges, strides, gt_bboxes, gt_labels):
    P = points.shape[0]
    B, G, _ = gt_bboxes.shape
    assert P % PB == 0 and B % 2 == 0 and 2 * G == LG
    npb = P // PB
    B2 = B // 2

    strides2 = strides.reshape(P, 1)

    gx = gt_bboxes[..., 0]
    gy = gt_bboxes[..., 1]
    gw = gt_bboxes[..., 2]
    gh = gt_bboxes[..., 3]
    gxe = gx + gw - 1.0
    gye = gy + gh - 1.0
    cx = gx + gw / 2.0
    cy = gy + gh / 2.0
    meas = jnp.maximum(gw, gh)
    rows = jnp.stack([gx, gy, gxe, gye, cx, cy, meas], axis=1)
    gt_pair = rows.reshape(B2, 2, 7, G).transpose(0, 2, 1, 3).reshape(
        B2, 7, LG)

    labf = gt_labels.astype(jnp.float32)
    gm = jnp.stack([gx, gy, gxe, gye, labf], axis=-1)
    gm2 = jnp.zeros((B2, LG, 16), jnp.float32)
    gm2 = gm2.at[:, 0:G, 0:5].set(gm[0::2])
    gm2 = gm2.at[:, G:LG, 8:13].set(gm[1::2])

    oh = (gt_labels[:, :, None] ==
          jnp.arange(NUM_CLASSES)[None, None, :]).astype(jnp.float32)
    oh2 = jnp.zeros((B2, LG, 256), jnp.float32)
    oh2 = oh2.at[:, 0:G, 0:NUM_CLASSES].set(oh[0::2])
    oh2 = oh2.at[:, G:LG, 128:128 + NUM_CLASSES].set(oh[1::2])

    cls, reg = pl.pallas_call(
        _assign_kernel,
        grid=(B2, npb),
        in_specs=[
            pl.BlockSpec((PB, 2), lambda b, i: (i, 0)),
            pl.BlockSpec((PB, 2), lambda b, i: (i, 0)),
            pl.BlockSpec((PB, 2), lambda b, i: (i, 0)),
            pl.BlockSpec((PB, 1), lambda b, i: (i, 0)),
            pl.BlockSpec((1, 7, LG), lambda b, i: (b, 0, 0)),
            pl.BlockSpec((1, LG, 16), lambda b, i: (b, 0, 0)),
            pl.BlockSpec((1, LG, 256), lambda b, i: (b, 0, 0)),
        ],
        out_specs=[
            pl.BlockSpec((2, PB, NUM_CLASSES), lambda b, i: (b, i, 0)),
            pl.BlockSpec((2, PB, 4), lambda b, i: (b, i, 0)),
        ],
        out_shape=[
            jax.ShapeDtypeStruct((B, P, NUM_CLASSES), jnp.float32),
            jax.ShapeDtypeStruct((B, P, 4), jnp.float32),
        ],
        compiler_params=pltpu.CompilerParams(
            dimension_semantics=("parallel", "arbitrary"),
        ),
    )(points, reg_ranges, gray_ranges, strides2, gt_pair, gm2, oh2)
    return cls, reg


# speedup vs baseline: 17.1966x; 1.0140x over previous
import jax
import jax.numpy as jnp
from jax.experimental import pallas as pl
from jax.experimental.pallas import tpu as pltpu

NUM_CLASSES = 80
PB = 1984
LG = 128


def _assign_kernel(pts_ref, rr_ref, gr_ref, st_ref, gt_ref, gm_ref, oh_ref,
                   cls_ref, reg_ref):
    pts = pts_ref[...]
    px = pts[:, 0:1]
    py = pts[:, 1:2]
    g = gt_ref[0]
    gx = g[0:1, :]
    gy = g[1:2, :]
    gxe = g[2:3, :]
    gye = g[3:4, :]
    cx = g[4:5, :]
    cy = g[5:6, :]
    meas = g[6:7, :]

    d1 = px - gx
    d2 = py - gy
    d3 = gxe - px
    d4 = gye - py
    hit = jnp.minimum(jnp.minimum(d1, d2), jnp.minimum(d3, d4)) >= 0.0
    hf = hit.astype(jnp.float32)
    f1 = d1 * hf
    f2 = d2 * hf
    f3 = d3 * hf
    f4 = d4 * hf
    q = (jnp.maximum(jnp.minimum(f1, f3), 0.0) /
         jnp.maximum(jnp.maximum(f1, f3), 0.01)) * \
        (jnp.maximum(jnp.minimum(f2, f4), 0.0) /
         jnp.maximum(jnp.maximum(f2, f4), 0.01))
    q = jnp.sqrt(q)

    s2 = st_ref[...] * 0.5
    inside_core = (px >= cx - s2) & (px <= cx + s2) & \
                  (py >= cy - s2) & (py <= cy + s2) & hit
    q = jnp.where(inside_core, 1.0, q)

    rr = rr_ref[...]
    gr = gr_ref[...]
    rlo = rr[:, 0:1]
    rhi = rr[:, 1:2]
    glo = gr[:, 0:1]
    ghi = gr[:, 1:2]
    green = (rlo <= meas) & (meas <= rhi) & hit
    gray = (((glo <= meas) & (meas < rlo)) |
            ((rhi < meas) & (meas <= ghi))) & hit
    q = q * green.astype(jnp.float32)

    PBn = q.shape[0]
    G = LG // 2
    qA = q[:, 0:G]
    qB = q[:, G:LG]
    maxA = jnp.max(qA, axis=1, keepdims=True)
    maxB = jnp.max(qB, axis=1, keepdims=True)
    maxfull = jnp.concatenate(
        [jnp.broadcast_to(maxA, (PBn, G)), jnp.broadcast_to(maxB, (PBn, G))],
        axis=1)
    gidx = jnp.bitwise_and(
        jax.lax.broadcasted_iota(jnp.int32, (PBn, LG), 1), G - 1)
    cand = jnp.where(q == maxfull, gidx, G)
    midxA = jnp.min(cand[:, 0:G], axis=1, keepdims=True)
    midxB = jnp.min(cand[:, G:LG], axis=1, keepdims=True)
    midxfull = jnp.concatenate(
        [jnp.broadcast_to(midxA, (PBn, G)), jnp.broadcast_to(midxB, (PBn, G))],
        axis=1)
    sel = (gidx == midxfull).astype(jnp.float32)

    t = jnp.dot(sel, gm_ref[0], preferred_element_type=jnp.float32)
    gcv = jnp.dot(gray.astype(jnp.float32), oh_ref[0],
                  preferred_element_type=jnp.float32)

    ciota = jax.lax.broadcasted_iota(jnp.int32, (PBn, NUM_CLASSES), 1)

    posA = maxA > 0.0
    valA = jnp.where(posA, maxA, 0.0)
    matchedA = t[:, 4:5].astype(jnp.int32)
    clsA = jnp.where(ciota == matchedA, valA, 0.0)
    grayA = gcv[:, 0:NUM_CLASSES] > 0.0
    clsA = jnp.where(grayA & (clsA == 0.0), -1.0, clsA)
    cls_ref[0] = clsA
    regA = jnp.concatenate(
        [px - t[:, 0:1], py - t[:, 1:2], t[:, 2:3] - px, t[:, 3:4] - py],
        axis=1) * posA.astype(jnp.float32)
    reg_ref[0] = regA

    posB = maxB > 0.0
    valB = jnp.where(posB, maxB, 0.0)
    matchedB = t[:, 12:13].astype(jnp.int32)
    clsB = jnp.where(ciota == matchedB, valB, 0.0)
    grayB = gcv[:, 128:128 + NUM_CLASSES] > 0.0
    clsB = jnp.where(grayB & (clsB == 0.0), -1.0, clsB)
    cls_ref[1] = clsB
    regB = jnp.concatenate(
        [px - t[:, 8:9], py - t[:, 9:10], t[:, 10:11] - px, t[:, 11:12] - py],
        axis=1) * posB.astype(jnp.float32)
    reg_ref[1] = regB


def kernel(points, reg_ranges, gray_ranges, strides, gt_bboxes, gt_labels):
    P = points.shape[0]
    B, G, _ = gt_bboxes.shape
    assert P % PB == 0 and B % 2 == 0 and 2 * G == LG
    npb = P // PB
    B2 = B // 2

    strides2 = strides.reshape(P, 1)

    gx = gt_bboxes[..., 0]
    gy = gt_bboxes[..., 1]
    gw = gt_bboxes[..., 2]
    gh = gt_bboxes[..., 3]
    gxe = gx + gw - 1.0
    gye = gy + gh - 1.0
    cx = gx + gw / 2.0
    cy = gy + gh / 2.0
    meas = jnp.maximum(gw, gh)
    rows = jnp.stack([gx, gy, gxe, gye, cx, cy, meas], axis=1)
    gt_pair = rows.reshape(B2, 2, 7, G).transpose(0, 2, 1, 3).reshape(
        B2, 7, LG)

    labf = gt_labels.astype(jnp.float32)
    gm = jnp.stack([gx, gy, gxe, gye, labf], axis=-1)
    gm2 = jnp.zeros((B2, LG, 16), jnp.float32)
    gm2 = gm2.at[:, 0:G, 0:5].set(gm[0::2])
    gm2 = gm2.at[:, G:LG, 8:13].set(gm[1::2])

    oh = (gt_labels[:, :, None] ==
          jnp.arange(NUM_CLASSES)[None, None, :]).astype(jnp.float32)
    oh2 = jnp.zeros((B2, LG, 256), jnp.float32)
    oh2 = oh2.at[:, 0:G, 0:NUM_CLASSES].set(oh[0::2])
    oh2 = oh2.at[:, G:LG, 128:128 + NUM_CLASSES].set(oh[1::2])

    cls, reg = pl.pallas_call(
        _assign_kernel,
        grid=(B2, npb),
        in_specs=[
            pl.BlockSpec((PB, 2), lambda b, i: (i, 0)),
            pl.BlockSpec((PB, 2), lambda b, i: (i, 0)),
            pl.BlockSpec((PB, 2), lambda b, i: (i, 0)),
            pl.BlockSpec((PB, 1), lambda b, i: (i, 0)),
            pl.BlockSpec((1, 7, LG), lambda b, i: (b, 0, 0)),
            pl.BlockSpec((1, LG, 16), lambda b, i: (b, 0, 0)),
            pl.BlockSpec((1, LG, 256), lambda b, i: (b, 0, 0)),
        ],
        out_specs=[
            pl.BlockSpec((2, PB, NUM_CLASSES), lambda b, i: (b, i, 0)),
            pl.BlockSpec((2, PB, 4), lambda b, i: (b, i, 0)),
        ],
        out_shape=[
            jax.ShapeDtypeStruct((B, P, NUM_CLASSES), jnp.float32),
            jax.ShapeDtypeStruct((B, P, 4), jnp.float32),
        ],
        compiler_params=pltpu.CompilerParams(
            dimension_semantics=("parallel", "arbitrary"),
        ),
    )(points, reg_ranges, gray_ranges, strides2, gt_pair, gm2, oh2)
    return cls, reg
